# baseline (speedup 1.0000x reference)
import dataclasses
import functools

import jax
import jax.numpy as jnp
from jax import lax
from jax.experimental import pallas as pl
from jax.experimental.pallas import tpu as pltpu
from jax.experimental.pallas import tpu_sc as plsc

N_NODES = 10000
D = 128
N_EDGES = 320000

NC, NS = 2, 16
NW = NC * NS
EPW = N_EDGES // NW
CHUNK = 80
NCHUNK = EPW // CHUNK
N_PAD = 10112
ROWS_PER_SUB = N_PAD // NS

_SELU_ALPHA = 1.6732632423543772
_SELU_SCALE = 1.0507009873554805

_ROW_BLK = 1000
REP = 4


def _linear_body(x_ref, w_ref, o_ref):
    res = lax.dot_general(
        x_ref[...], w_ref[...], (((1,), (1,)), ((), ())),
        preferred_element_type=jnp.float32)
    o_ref[...] = jnp.broadcast_to(res[None], (REP, _ROW_BLK, D))


def _linear(features, W):
    return pl.pallas_call(
        _linear_body,
        grid=(N_NODES // _ROW_BLK,),
        in_specs=[
            pl.BlockSpec((_ROW_BLK, D), lambda i: (i, 0)),
            pl.BlockSpec((D, D), lambda i: (0, 0)),
        ],
        out_specs=pl.BlockSpec((REP, _ROW_BLK, D), lambda i: (0, i, 0)),
        out_shape=jax.ShapeDtypeStruct((REP, N_NODES, D), jnp.float32),
    )(features, W)


def _sc_aggregate(out, src, dst, ev, zeros):
    mesh = plsc.VectorSubcoreMesh(core_axis_name="c", subcore_axis_name="s")
    cp = pltpu.CompilerParams()
    if "needs_layout_passes" in pltpu.CompilerParams.__dataclass_fields__:
        cp = dataclasses.replace(cp, needs_layout_passes=False)

    @functools.partial(
        pl.kernel,
        compiler_params=cp,
        out_type=jax.ShapeDtypeStruct((NC, N_PAD, D), jnp.float32),
        mesh=mesh,
        scratch_types=[
            pltpu.VMEM((NCHUNK, CHUNK), jnp.int32),
            pltpu.VMEM((1, CHUNK), jnp.float32),
            pltpu.VMEM((1, CHUNK), jnp.float32),
            pltpu.VMEM((1, CHUNK), jnp.int32),
            pltpu.VMEM((1, CHUNK), jnp.int32),
            pltpu.VMEM((CHUNK, D), jnp.float32),
            pltpu.VMEM((CHUNK, D), jnp.float32),
            pltpu.VMEM_SHARED((N_PAD, D), jnp.float32),
            pltpu.SemaphoreType.DMA,
            pltpu.SemaphoreType.DMA,
            pltpu.SemaphoreType.DMA,
            pltpu.SemaphoreType.DMA,
            pltpu.SemaphoreType.DMA,
            pltpu.SemaphoreType.DMA,
            pltpu.SemaphoreType.DMA,
            pltpu.SemaphoreType.DMA,
            pltpu.SemaphoreType.DMA,
            pltpu.SemaphoreType.DMA,
        ],
    )
    def k(out_hbm, src_hbm, dst_hbm, ev_hbm, zeros_hbm, agg_hbm,
          sidx_v, evb_a, evb_b, didx_a, didx_b, rows_a, rows_b, acc_sh,
          gsem_a0, gsem_b0, gsem_a1, gsem_b1,
          ssem_a, ssem_b, dsem_a, dsem_b, esem_a, esem_b):
        cid = lax.axis_index("c")
        sid = lax.axis_index("s")
        wid = sid * NC + cid

        rows = (rows_a, rows_b)
        didx = (didx_a, didx_b)
        evb = (evb_a, evb_b)
        gsem = ((gsem_a0, gsem_a1), (gsem_b0, gsem_b1))
        ssem = (ssem_a, ssem_b)
        dsem = (dsem_a, dsem_b)
        esem = (esem_a, esem_b)

        pltpu.sync_copy(
            zeros_hbm.at[pl.ds(sid * ROWS_PER_SUB, ROWS_PER_SUB)],
            acc_sh.at[pl.ds(sid * ROWS_PER_SUB, ROWS_PER_SUB)])

        pltpu.sync_copy(src_hbm.at[wid], sidx_v)
        plsc.subcore_barrier()

        half = CHUNK // 2

        def gather_start(c, p):
            for h in range(2):
                sl = pl.ds(h * half, half)
                pltpu.async_copy(out_hbm.at[sidx_v.at[c, sl]],
                                 rows[p].at[sl], gsem[p][h])

        def gather_wait(c, p):
            for h in range(2):
                sl = pl.ds(h * half, half)
                pltpu.make_async_copy(out_hbm.at[sidx_v.at[c, sl]],
                                      rows[p].at[sl], gsem[p][h]).wait()

        def scatter_start(c, p):
            pltpu.async_copy(rows[p], acc_sh.at[didx[p].at[0]], ssem[p],
                             add=True)

        def scatter_wait(c, p):
            pltpu.make_async_copy(
                rows[p], acc_sh.at[didx[p].at[0]], ssem[p]).wait()

        def dfetch_start(c, p):
            pltpu.async_copy(
                dst_hbm.at[wid, pl.ds(c, 1)], didx[p], dsem[p])
            pltpu.async_copy(
                ev_hbm.at[wid, pl.ds(c, 1)], evb[p], esem[p])

        def dfetch_wait(c, p):
            pltpu.make_async_copy(
                dst_hbm.at[wid, pl.ds(c, 1)], didx[p], dsem[p]).wait()

        def efetch_wait(c, p):
            pltpu.make_async_copy(
                ev_hbm.at[wid, pl.ds(c, 1)], evb[p], esem[p]).wait()

        def scale(c, p):
            rp = rows[p]
            ep = evb[p]

            @plsc.parallel_loop(0, CHUNK // 16, unroll=1)
            def _grp(g):
                ev16 = ep[0, pl.ds(g * 16, 16)]
                for j in range(16):
                    evj = lax.gather(
                        ev16, jnp.full((16, 1), j, jnp.int32),
                        lax.GatherDimensionNumbers(
                            offset_dims=(), collapsed_slice_dims=(0,),
                            start_index_map=(0,)),
                        slice_sizes=(1,),
                        mode=lax.GatherScatterMode.PROMISE_IN_BOUNDS)
                    e = g * 16 + j
                    for kk in range(D // 16):
                        sl = pl.ds(kk * 16, 16)
                        rp[e, sl] = rp[e, sl] * evj

        gather_start(0, 0)
        dfetch_start(0, 0)

        @pl.loop(0, NCHUNK - 1, step=2)
        def _pair(c0):
            for p in (0, 1):
                c = c0 + p
                gather_wait(c, p)
                if p == 0:
                    @pl.when(c0 > 0)
                    def _():
                        scatter_wait(c - 1, 1 - p)
                else:
                    scatter_wait(c - 1, 1 - p)
                gather_start(c + 1, 1 - p)
                dfetch_start(c + 1, 1 - p)
                efetch_wait(c, p)
                scale(c, p)
                dfetch_wait(c, p)
                scatter_start(c, p)

        c_last = NCHUNK - 1
        gather_wait(c_last, 0)
        scatter_wait(c_last - 1, 1)
        efetch_wait(c_last, 0)
        scale(c_last, 0)
        dfetch_wait(c_last, 0)
        scatter_start(c_last, 0)
        scatter_wait(c_last, 0)

        plsc.subcore_barrier()
        pltpu.sync_copy(
            acc_sh.at[pl.ds(sid * ROWS_PER_SUB, ROWS_PER_SUB)],
            agg_hbm.at[cid, pl.ds(sid * ROWS_PER_SUB, ROWS_PER_SUB)])

    return k(out, src, dst, ev, zeros)


def _combine_body(o_ref, a0_ref, a1_ref, sw_ref, b_ref, y_ref):
    x = (o_ref[0] * sw_ref[...] + a0_ref[...] + a1_ref[...] + b_ref[...])
    y_ref[...] = _SELU_SCALE * jnp.where(
        x > 0, x, _SELU_ALPHA * (jnp.exp(x) - 1.0))


def _combine(out_rep, a0, a1, skip_weight, bias):
    blk = pl.BlockSpec((_ROW_BLK, D), lambda i: (i, 0))
    vec = pl.BlockSpec((1, D), lambda i: (0, 0))
    return pl.pallas_call(
        _combine_body,
        grid=(N_NODES // _ROW_BLK,),
        in_specs=[pl.BlockSpec((1, _ROW_BLK, D), lambda i: (0, i, 0)),
                  blk, blk, vec, vec],
        out_specs=blk,
        out_shape=jax.ShapeDtypeStruct((N_NODES, D), jnp.float32),
    )(out_rep, a0, a1, skip_weight, bias)


def kernel(features, edge_index, edge_values, W, skip_weight, bias):
    out_rep = _linear(features, W)
    src = edge_index[0].reshape(NW, NCHUNK, CHUNK)
    rep_off = (jnp.arange(NW, dtype=src.dtype) % REP) * N_NODES
    src = src + rep_off[:, None, None]
    dst = edge_index[1].reshape(NW, NCHUNK, CHUNK)
    edge_values = edge_values.reshape(NW, NCHUNK, CHUNK)
    zeros = jnp.zeros((N_PAD, D), jnp.float32)
    out_flat = out_rep.reshape(REP * N_NODES, D)
    agg = _sc_aggregate(out_flat, src, dst, edge_values, zeros)[:, :N_NODES]
    return _combine(out_rep, agg[0], agg[1],
                    skip_weight.reshape(1, D), bias.reshape(1, D))


# speedup vs baseline: 1.0004x; 1.0004x over previous
import dataclasses
import functools

import jax
import jax.numpy as jnp
from jax import lax
from jax.experimental import pallas as pl
from jax.experimental.pallas import tpu as pltpu
from jax.experimental.pallas import tpu_sc as plsc

N_NODES = 10000
D = 128
N_EDGES = 320000

NC, NS = 2, 16
NW = NC * NS
EPW = N_EDGES // NW
CHUNK = 80
NCHUNK = EPW // CHUNK
N_PAD = 10112
ROWS_PER_SUB = N_PAD // NS

_SELU_ALPHA = 1.6732632423543772
_SELU_SCALE = 1.0507009873554805

_ROW_BLK = 1000


def _linear_body(x_ref, w_ref, o_ref):
    o_ref[...] = lax.dot_general(
        x_ref[...], w_ref[...], (((1,), (1,)), ((), ())),
        preferred_element_type=jnp.float32)


def _linear(features, W):
    return pl.pallas_call(
        _linear_body,
        grid=(N_NODES // _ROW_BLK,),
        in_specs=[
            pl.BlockSpec((_ROW_BLK, D), lambda i: (i, 0)),
            pl.BlockSpec((D, D), lambda i: (0, 0)),
        ],
        out_specs=pl.BlockSpec((_ROW_BLK, D), lambda i: (i, 0)),
        out_shape=jax.ShapeDtypeStruct((N_NODES, D), jnp.float32),
    )(features, W)


def _sc_aggregate(out, src, dst, ev, zeros):
    mesh = plsc.VectorSubcoreMesh(core_axis_name="c", subcore_axis_name="s")
    cp = pltpu.CompilerParams()
    if "needs_layout_passes" in pltpu.CompilerParams.__dataclass_fields__:
        cp = dataclasses.replace(cp, needs_layout_passes=False)

    @functools.partial(
        pl.kernel,
        compiler_params=cp,
        out_type=jax.ShapeDtypeStruct((NC, N_PAD, D), jnp.float32),
        mesh=mesh,
        scratch_types=[
            pltpu.VMEM((NCHUNK, CHUNK), jnp.int32),
            pltpu.VMEM((1, CHUNK), jnp.float32),
            pltpu.VMEM((1, CHUNK), jnp.float32),
            pltpu.VMEM((1, CHUNK), jnp.int32),
            pltpu.VMEM((1, CHUNK), jnp.int32),
            pltpu.VMEM((CHUNK, D), jnp.float32),
            pltpu.VMEM((CHUNK, D), jnp.float32),
            pltpu.VMEM_SHARED((N_PAD, D), jnp.float32),
            pltpu.SemaphoreType.DMA,
            pltpu.SemaphoreType.DMA,
            pltpu.SemaphoreType.DMA,
            pltpu.SemaphoreType.DMA,
            pltpu.SemaphoreType.DMA,
            pltpu.SemaphoreType.DMA,
            pltpu.SemaphoreType.DMA,
            pltpu.SemaphoreType.DMA,
            pltpu.SemaphoreType.DMA,
            pltpu.SemaphoreType.DMA,
            pltpu.SemaphoreType.DMA,
            pltpu.SemaphoreType.DMA,
            pltpu.SemaphoreType.DMA,
            pltpu.SemaphoreType.DMA,
        ],
    )
    def k(out_hbm, src_hbm, dst_hbm, ev_hbm, zeros_hbm, agg_hbm,
          sidx_v, evb_a, evb_b, didx_a, didx_b, rows_a, rows_b, acc_sh,
          gsem_a0, gsem_b0, gsem_a1, gsem_b1,
          gsem_a2, gsem_b2, gsem_a3, gsem_b3,
          ssem_a, ssem_b, dsem_a, dsem_b, esem_a, esem_b):
        cid = lax.axis_index("c")
        sid = lax.axis_index("s")
        wid = sid * NC + cid

        rows = (rows_a, rows_b)
        didx = (didx_a, didx_b)
        evb = (evb_a, evb_b)
        gsem = ((gsem_a0, gsem_a1, gsem_a2, gsem_a3),
                (gsem_b0, gsem_b1, gsem_b2, gsem_b3))
        ssem = (ssem_a, ssem_b)
        dsem = (dsem_a, dsem_b)
        esem = (esem_a, esem_b)

        pltpu.sync_copy(src_hbm.at[wid], sidx_v)

        nsub = 4
        sub = CHUNK // nsub

        def gather_start(c, p):
            for h in range(nsub):
                sl = pl.ds(h * sub, sub)
                pltpu.async_copy(out_hbm.at[sidx_v.at[c, sl]],
                                 rows[p].at[sl], gsem[p][h])

        def gather_wait(c, p):
            for h in range(nsub):
                sl = pl.ds(h * sub, sub)
                pltpu.make_async_copy(out_hbm.at[sidx_v.at[c, sl]],
                                      rows[p].at[sl], gsem[p][h]).wait()

        def scatter_start(c, p):
            pltpu.async_copy(rows[p], acc_sh.at[didx[p].at[0]], ssem[p],
                             add=True)

        def scatter_wait(c, p):
            pltpu.make_async_copy(
                rows[p], acc_sh.at[didx[p].at[0]], ssem[p]).wait()

        def dfetch_start(c, p):
            pltpu.async_copy(
                dst_hbm.at[wid, pl.ds(c, 1)], didx[p], dsem[p])
            pltpu.async_copy(
                ev_hbm.at[wid, pl.ds(c, 1)], evb[p], esem[p])

        def dfetch_wait(c, p):
            pltpu.make_async_copy(
                dst_hbm.at[wid, pl.ds(c, 1)], didx[p], dsem[p]).wait()

        def efetch_wait(c, p):
            pltpu.make_async_copy(
                ev_hbm.at[wid, pl.ds(c, 1)], evb[p], esem[p]).wait()

        def scale(c, p):
            rp = rows[p]
            ep = evb[p]

            @plsc.parallel_loop(0, CHUNK // 16, unroll=1)
            def _grp(g):
                ev16 = ep[0, pl.ds(g * 16, 16)]
                for j in range(16):
                    evj = lax.gather(
                        ev16, jnp.full((16, 1), j, jnp.int32),
                        lax.GatherDimensionNumbers(
                            offset_dims=(), collapsed_slice_dims=(0,),
                            start_index_map=(0,)),
                        slice_sizes=(1,),
                        mode=lax.GatherScatterMode.PROMISE_IN_BOUNDS)
                    e = g * 16 + j
                    for kk in range(D // 16):
                        sl = pl.ds(kk * 16, 16)
                        rp[e, sl] = rp[e, sl] * evj

        gather_start(0, 0)
        dfetch_start(0, 0)

        pltpu.sync_copy(
            zeros_hbm.at[pl.ds(sid * ROWS_PER_SUB, ROWS_PER_SUB)],
            acc_sh.at[pl.ds(sid * ROWS_PER_SUB, ROWS_PER_SUB)])
        plsc.subcore_barrier()

        @pl.loop(0, NCHUNK - 1, step=2)
        def _pair(c0):
            for p in (0, 1):
                c = c0 + p
                gather_wait(c, p)
                if p == 0:
                    @pl.when(c0 > 0)
                    def _():
                        scatter_wait(c - 1, 1 - p)
                else:
                    scatter_wait(c - 1, 1 - p)
                gather_start(c + 1, 1 - p)
                dfetch_start(c + 1, 1 - p)
                efetch_wait(c, p)
                scale(c, p)
                dfetch_wait(c, p)
                scatter_start(c, p)

        c_last = NCHUNK - 1
        gather_wait(c_last, 0)
        scatter_wait(c_last - 1, 1)
        efetch_wait(c_last, 0)
        scale(c_last, 0)
        dfetch_wait(c_last, 0)
        scatter_start(c_last, 0)
        scatter_wait(c_last, 0)

        plsc.subcore_barrier()
        pltpu.sync_copy(
            acc_sh.at[pl.ds(sid * ROWS_PER_SUB, ROWS_PER_SUB)],
            agg_hbm.at[cid, pl.ds(sid * ROWS_PER_SUB, ROWS_PER_SUB)])

    return k(out, src, dst, ev, zeros)


def _combine_body(o_ref, a0_ref, a1_ref, sw_ref, b_ref, y_ref):
    x = (o_ref[...] * sw_ref[...] + a0_ref[...] + a1_ref[...] + b_ref[...])
    y_ref[...] = _SELU_SCALE * jnp.where(
        x > 0, x, _SELU_ALPHA * (jnp.exp(x) - 1.0))


def _combine(out, a0, a1, skip_weight, bias):
    blk = pl.BlockSpec((_ROW_BLK, D), lambda i: (i, 0))
    vec = pl.BlockSpec((1, D), lambda i: (0, 0))
    return pl.pallas_call(
        _combine_body,
        grid=(N_NODES // _ROW_BLK,),
        in_specs=[blk, blk, blk, vec, vec],
        out_specs=blk,
        out_shape=jax.ShapeDtypeStruct((N_NODES, D), jnp.float32),
    )(out, a0, a1, skip_weight, bias)


def kernel(features, edge_index, edge_values, W, skip_weight, bias):
    out = _linear(features, W)
    src = edge_index[0].reshape(NW, NCHUNK, CHUNK)
    dst = edge_index[1].reshape(NW, NCHUNK, CHUNK)
    edge_values = edge_values.reshape(NW, NCHUNK, CHUNK)
    zeros = jnp.zeros((N_PAD, D), jnp.float32)
    agg = _sc_aggregate(out, src, dst, edge_values, zeros)[:, :N_NODES]
    return _combine(out, agg[0], agg[1],
                    skip_weight.reshape(1, D), bias.reshape(1, D))


# speedup vs baseline: 1.0193x; 1.0189x over previous
import dataclasses
import functools

import jax
import jax.numpy as jnp
from jax import lax
from jax.experimental import pallas as pl
from jax.experimental.pallas import tpu as pltpu
from jax.experimental.pallas import tpu_sc as plsc

N_NODES = 10000
D = 128
N_EDGES = 320000

NC, NS = 2, 16
NW = NC * NS
EPW = N_EDGES // NW
CHUNK = 80
NCHUNK = EPW // CHUNK
N_PAD = 10112
ROWS_PER_SUB = N_PAD // NS

_SELU_ALPHA = 1.6732632423543772
_SELU_SCALE = 1.0507009873554805

_ROW_BLK = 1000


def _linear_body(x_ref, w_ref, o_ref):
    o_ref[...] = lax.dot_general(
        x_ref[...], w_ref[...], (((1,), (1,)), ((), ())),
        preferred_element_type=jnp.float32)


def _linear(features, W):
    return pl.pallas_call(
        _linear_body,
        grid=(N_NODES // _ROW_BLK,),
        in_specs=[
            pl.BlockSpec((_ROW_BLK, D), lambda i: (i, 0)),
            pl.BlockSpec((D, D), lambda i: (0, 0)),
        ],
        out_specs=pl.BlockSpec((_ROW_BLK, D), lambda i: (i, 0)),
        out_shape=jax.ShapeDtypeStruct((N_NODES, D), jnp.float32),
    )(features, W)


def _sc_aggregate(out, src, dst, ev, zeros):
    mesh = plsc.VectorSubcoreMesh(core_axis_name="c", subcore_axis_name="s")
    cp = pltpu.CompilerParams()
    if "needs_layout_passes" in pltpu.CompilerParams.__dataclass_fields__:
        cp = dataclasses.replace(cp, needs_layout_passes=False)

    @functools.partial(
        pl.kernel,
        compiler_params=cp,
        out_type=jax.ShapeDtypeStruct((NC, N_PAD, D), jnp.float32),
        mesh=mesh,
        scratch_types=[
            pltpu.VMEM((NCHUNK, CHUNK), jnp.int32),
            pltpu.VMEM((1, CHUNK), jnp.float32),
            pltpu.VMEM((1, CHUNK), jnp.float32),
            pltpu.VMEM((1, CHUNK), jnp.int32),
            pltpu.VMEM((1, CHUNK), jnp.int32),
            pltpu.VMEM((CHUNK, D), jnp.float32),
            pltpu.VMEM((CHUNK, D), jnp.float32),
            pltpu.VMEM_SHARED((N_PAD, D), jnp.float32),
            pltpu.SemaphoreType.DMA,
            pltpu.SemaphoreType.DMA,
            pltpu.SemaphoreType.DMA,
            pltpu.SemaphoreType.DMA,
            pltpu.SemaphoreType.DMA,
            pltpu.SemaphoreType.DMA,
            pltpu.SemaphoreType.DMA,
            pltpu.SemaphoreType.DMA,
            pltpu.SemaphoreType.DMA,
            pltpu.SemaphoreType.DMA,
        ],
    )
    def k(out_hbm, src_hbm, dst_hbm, ev_hbm, zeros_hbm, agg_hbm,
          sidx_v, evb_a, evb_b, didx_a, didx_b, rows_a, rows_b, acc_sh,
          gsem_a0, gsem_b0, gsem_a1, gsem_b1,
          ssem_a, ssem_b, dsem_a, dsem_b, esem_a, esem_b):
        cid = lax.axis_index("c")
        sid = lax.axis_index("s")
        wid = sid * NC + cid

        rows = (rows_a, rows_b)
        didx = (didx_a, didx_b)
        evb = (evb_a, evb_b)
        gsem = ((gsem_a0, gsem_a1), (gsem_b0, gsem_b1))
        ssem = (ssem_a, ssem_b)
        dsem = (dsem_a, dsem_b)
        esem = (esem_a, esem_b)

        pltpu.sync_copy(src_hbm.at[wid], sidx_v)

        nsub = 2
        sub = CHUNK // nsub

        def gather_start(c, p):
            for h in range(nsub):
                sl = pl.ds(h * sub, sub)
                pltpu.async_copy(out_hbm.at[sidx_v.at[c, sl]],
                                 rows[p].at[sl], gsem[p][h])

        def gather_wait(c, p):
            for h in range(nsub):
                sl = pl.ds(h * sub, sub)
                pltpu.make_async_copy(out_hbm.at[sidx_v.at[c, sl]],
                                      rows[p].at[sl], gsem[p][h]).wait()

        def scatter_start(c, p):
            pltpu.async_copy(rows[p], acc_sh.at[didx[p].at[0]], ssem[p],
                             add=True)

        def scatter_wait(c, p):
            pltpu.make_async_copy(
                rows[p], acc_sh.at[didx[p].at[0]], ssem[p]).wait()

        def dfetch_start(c, p):
            pltpu.async_copy(
                dst_hbm.at[wid, pl.ds(c, 1)], didx[p], dsem[p])
            pltpu.async_copy(
                ev_hbm.at[wid, pl.ds(c, 1)], evb[p], esem[p])

        def dfetch_wait(c, p):
            pltpu.make_async_copy(
                dst_hbm.at[wid, pl.ds(c, 1)], didx[p], dsem[p]).wait()

        def efetch_wait(c, p):
            pltpu.make_async_copy(
                ev_hbm.at[wid, pl.ds(c, 1)], evb[p], esem[p]).wait()

        def scale(c, p):
            rp = rows[p]
            ep = evb[p]

            @plsc.parallel_loop(0, CHUNK // 16, unroll=1)
            def _grp(g):
                ev16 = ep[0, pl.ds(g * 16, 16)]
                for j in range(16):
                    evj = lax.gather(
                        ev16, jnp.full((16, 1), j, jnp.int32),
                        lax.GatherDimensionNumbers(
                            offset_dims=(), collapsed_slice_dims=(0,),
                            start_index_map=(0,)),
                        slice_sizes=(1,),
                        mode=lax.GatherScatterMode.PROMISE_IN_BOUNDS)
                    e = g * 16 + j
                    for kk in range(D // 16):
                        sl = pl.ds(kk * 16, 16)
                        rp[e, sl] = rp[e, sl] * evj

        gather_start(0, 0)
        dfetch_start(0, 0)

        pltpu.sync_copy(
            zeros_hbm.at[pl.ds(sid * ROWS_PER_SUB, ROWS_PER_SUB)],
            acc_sh.at[pl.ds(sid * ROWS_PER_SUB, ROWS_PER_SUB)])
        plsc.subcore_barrier()

        @pl.loop(0, NCHUNK - 1, step=2)
        def _pair(c0):
            for p in (0, 1):
                c = c0 + p
                gather_wait(c, p)
                if p == 0:
                    @pl.when(c0 > 0)
                    def _():
                        scatter_wait(c - 1, 1 - p)
                else:
                    scatter_wait(c - 1, 1 - p)
                gather_start(c + 1, 1 - p)
                dfetch_start(c + 1, 1 - p)
                efetch_wait(c, p)
                scale(c, p)
                dfetch_wait(c, p)
                scatter_start(c, p)

        c_last = NCHUNK - 1
        gather_wait(c_last, 0)
        scatter_wait(c_last - 1, 1)
        efetch_wait(c_last, 0)
        scale(c_last, 0)
        dfetch_wait(c_last, 0)
        scatter_start(c_last, 0)
        scatter_wait(c_last, 0)

        plsc.subcore_barrier()
        pltpu.sync_copy(
            acc_sh.at[pl.ds(sid * ROWS_PER_SUB, ROWS_PER_SUB)],
            agg_hbm.at[cid, pl.ds(sid * ROWS_PER_SUB, ROWS_PER_SUB)])

    return k(out, src, dst, ev, zeros)


def _combine_body(o_ref, a0_ref, a1_ref, sw_ref, b_ref, y_ref):
    x = (o_ref[...] * sw_ref[...] + a0_ref[...] + a1_ref[...] + b_ref[...])
    y_ref[...] = _SELU_SCALE * jnp.where(
        x > 0, x, _SELU_ALPHA * (jnp.exp(x) - 1.0))


def _combine(out, a0, a1, skip_weight, bias):
    blk = pl.BlockSpec((_ROW_BLK, D), lambda i: (i, 0))
    vec = pl.BlockSpec((1, D), lambda i: (0, 0))
    return pl.pallas_call(
        _combine_body,
        grid=(N_NODES // _ROW_BLK,),
        in_specs=[blk, blk, blk, vec, vec],
        out_specs=blk,
        out_shape=jax.ShapeDtypeStruct((N_NODES, D), jnp.float32),
    )(out, a0, a1, skip_weight, bias)


def kernel(features, edge_index, edge_values, W, skip_weight, bias):
    out = _linear(features, W)
    src = edge_index[0].reshape(NW, NCHUNK, CHUNK)
    dst = edge_index[1].reshape(NW, NCHUNK, CHUNK)
    edge_values = edge_values.reshape(NW, NCHUNK, CHUNK)
    zeros = jnp.zeros((N_PAD, D), jnp.float32)
    agg = _sc_aggregate(out, src, dst, edge_values, zeros)[:, :N_NODES]
    return _combine(out, agg[0], agg[1],
                    skip_weight.reshape(1, D), bias.reshape(1, D))
